# Initial kernel scaffold; baseline (speedup 1.0000x reference)
import jax
import jax.numpy as jnp
from jax.experimental import pallas as pl


def kernel(x, H, dv_inv, de_inv, weight, bias):
    raise NotImplementedError("write your pallas kernel here")


# speedup vs baseline: 1.0676x; 1.0676x over previous
import functools

import jax
import jax.numpy as jnp
from jax.experimental import pallas as pl
from jax.experimental.pallas import tpu as pltpu


def _hgnn_kernel(x_ref, w_ref, b_ref, dv_ref, de_ref, h_ref, out_ref,
                 xnorm_ref, acc_ref, *, num_blocks):
    i = pl.program_id(0)

    @pl.when(i == 0)
    def _prologue():
        xm = jnp.dot(x_ref[...], w_ref[...],
                     preferred_element_type=jnp.float32) + b_ref[...]
        out_ref[...] = xm
        xnorm_ref[...] = xm * dv_ref[...]
        acc_ref[...] = jnp.zeros_like(acc_ref)

    h = h_ref[...]
    e = jax.lax.dot_general(
        h, xnorm_ref[...],
        dimension_numbers=(((0,), (0,)), ((), ())),
        preferred_element_type=jnp.float32)
    e2 = e * de_ref[...].T
    acc_ref[...] += jnp.dot(h, e2, preferred_element_type=jnp.float32)

    @pl.when(i == num_blocks - 1)
    def _epilogue():
        out_ref[...] = acc_ref[...] * dv_ref[...] + out_ref[...]


@jax.jit
def kernel(x, H, dv_inv, de_inv, weight, bias):
    N, d_in = x.shape
    M = H.shape[1]
    d_out = weight.shape[1]

    Mb = 256
    while M % Mb != 0:
        Mb //= 2
    num_blocks = M // Mb

    dv2 = dv_inv.reshape(N, 1)
    de2 = de_inv.reshape(1, M)
    b2 = bias.reshape(1, d_out)

    out = pl.pallas_call(
        functools.partial(_hgnn_kernel, num_blocks=num_blocks),
        grid=(num_blocks,),
        in_specs=[
            pl.BlockSpec((N, d_in), lambda i: (0, 0)),
            pl.BlockSpec((d_in, d_out), lambda i: (0, 0)),
            pl.BlockSpec((1, d_out), lambda i: (0, 0)),
            pl.BlockSpec((N, 1), lambda i: (0, 0)),
            pl.BlockSpec((1, Mb), lambda i: (0, i)),
            pl.BlockSpec((N, Mb), lambda i: (0, i)),
        ],
        out_specs=pl.BlockSpec((N, d_out), lambda i: (0, 0)),
        out_shape=jax.ShapeDtypeStruct((N, d_out), jnp.float32),
        scratch_shapes=[
            pltpu.VMEM((N, d_out), jnp.float32),
            pltpu.VMEM((N, d_out), jnp.float32),
        ],
        compiler_params=pltpu.CompilerParams(
            dimension_semantics=("arbitrary",),
            vmem_limit_bytes=110 * 1024 * 1024,
        ),
    )(x, weight, b2, dv2, de2, H)
    return out


# speedup vs baseline: 1.5268x; 1.4300x over previous
import functools

import jax
import jax.numpy as jnp
from jax.experimental import pallas as pl
from jax.experimental.pallas import tpu as pltpu


def _hgnn_kernel(x_ref, w_ref, b_ref, dv_ref, de_ref, h_ref, out_ref,
                 xnorm_ref, acc_ref, *, num_blocks):
    i = pl.program_id(0)

    @pl.when(i == 0)
    def _prologue():
        xm = jnp.dot(x_ref[...], w_ref[...],
                     preferred_element_type=jnp.float32) + b_ref[...]
        out_ref[...] = xm
        xnorm_ref[...] = xm * dv_ref[...]
        acc_ref[...] = jnp.zeros_like(acc_ref)

    h = h_ref[...].astype(jnp.bfloat16)
    e = jax.lax.dot_general(
        h, xnorm_ref[...].astype(jnp.bfloat16),
        dimension_numbers=(((0,), (0,)), ((), ())),
        preferred_element_type=jnp.float32)
    e2 = (e * de_ref[...].T).astype(jnp.bfloat16)
    acc_ref[...] += jnp.dot(h, e2, preferred_element_type=jnp.float32)

    @pl.when(i == num_blocks - 1)
    def _epilogue():
        out_ref[...] = acc_ref[...] * dv_ref[...] + out_ref[...]


@jax.jit
def kernel(x, H, dv_inv, de_inv, weight, bias):
    N, d_in = x.shape
    M = H.shape[1]
    d_out = weight.shape[1]

    Mb = 256
    while M % Mb != 0:
        Mb //= 2
    num_blocks = M // Mb

    dv2 = dv_inv.reshape(N, 1)
    de2 = de_inv.reshape(1, M)
    b2 = bias.reshape(1, d_out)

    out = pl.pallas_call(
        functools.partial(_hgnn_kernel, num_blocks=num_blocks),
        grid=(num_blocks,),
        in_specs=[
            pl.BlockSpec((N, d_in), lambda i: (0, 0)),
            pl.BlockSpec((d_in, d_out), lambda i: (0, 0)),
            pl.BlockSpec((1, d_out), lambda i: (0, 0)),
            pl.BlockSpec((N, 1), lambda i: (0, 0)),
            pl.BlockSpec((1, Mb), lambda i: (0, i)),
            pl.BlockSpec((N, Mb), lambda i: (0, i)),
        ],
        out_specs=pl.BlockSpec((N, d_out), lambda i: (0, 0)),
        out_shape=jax.ShapeDtypeStruct((N, d_out), jnp.float32),
        scratch_shapes=[
            pltpu.VMEM((N, d_out), jnp.float32),
            pltpu.VMEM((N, d_out), jnp.float32),
        ],
        compiler_params=pltpu.CompilerParams(
            dimension_semantics=("arbitrary",),
            vmem_limit_bytes=110 * 1024 * 1024,
        ),
    )(x, weight, b2, dv2, de2, H)
    return out
